# baseline (speedup 1.0000x reference)
import jax
import jax.numpy as jnp
from jax import lax
from jax.experimental import pallas as pl
from jax.experimental.pallas import tpu as pltpu
from jax.experimental.pallas import tpu_sc as plsc

VOCAB = 100000
EMBED_DIM = 64
B = 4096
T = 200
N = B * T

NC = 2
NS = 16
NW = NC * NS

PER_B = B // NW
L = 16
FLAT = PER_B * EMBED_DIM


def _fire_gather(table_hbm, idx_v, rows, sem, t):
    pltpu.async_copy(table_hbm.at[idx_v.at[t]], rows, sem)


def _wait_rows(table_hbm, rows, sem):
    pltpu.make_async_copy(table_hbm.at[pl.ds(0, PER_B)], rows, sem).wait()


def _fire_out(out_hbm, tbuf, sem, b0, t):
    pltpu.async_copy(tbuf, out_hbm.at[t, :, pl.ds(b0, PER_B)], sem)


def _wait_out(out_hbm, tbuf, sem):
    pltpu.make_async_copy(tbuf, out_hbm.at[0, :, pl.ds(0, PER_B)], sem).wait()


def _transpose(rows, tbuf, iota64x):
    iota16 = iota64x
    for jb in range(PER_B // L):
        ridx = iota16 + (L * jb)
        for d in range(EMBED_DIM):
            v = plsc.load_gather(rows, [ridx, jnp.full((L,), d, jnp.int32)])
            tbuf[d, pl.ds(jb * L, L)] = v


def _body(idxt_hbm, table_hbm, out_hbm,
          idx_v, rows0, rows1, tb0, tb1, g0, g1, o0, o1):
    wid = lax.axis_index("s") * NC + lax.axis_index("c")
    b0 = wid * PER_B
    rows = (rows0, rows1)
    tbuf = (tb0, tb1)
    gsem = (g0, g1)
    osem = (o0, o1)
    iota64x = lax.iota(jnp.int32, L)

    pltpu.sync_copy(idxt_hbm.at[:, pl.ds(b0, PER_B)], idx_v)

    _fire_gather(table_hbm, idx_v, rows[0], gsem[0], 0)
    _fire_gather(table_hbm, idx_v, rows[1], gsem[1], 1)

    def slot(t, b):
        @pl.when(t >= 2)
        def _():
            _wait_out(out_hbm, tbuf[b], osem[b])

        _wait_rows(table_hbm, rows[b], gsem[b])
        _transpose(rows[b], tbuf[b], iota64x)
        _fire_out(out_hbm, tbuf[b], osem[b], b0, t)

        @pl.when(t + 2 < T)
        def _():
            _fire_gather(table_hbm, idx_v, rows[b], gsem[b], t + 2)

    def pair(g, carry):
        slot(2 * g, 0)
        slot(2 * g + 1, 1)
        return carry

    lax.fori_loop(0, T // 2, pair, 0)

    _wait_out(out_hbm, tbuf[0], osem[0])
    _wait_out(out_hbm, tbuf[1], osem[1])


def _gather_sc(idx_t, table):
    mesh = plsc.VectorSubcoreMesh(core_axis_name="c", subcore_axis_name="s")
    kern = pl.kernel(
        _body,
        out_type=jax.ShapeDtypeStruct((T, EMBED_DIM, B), jnp.float32),
        mesh=mesh,
        scratch_types=[
            pltpu.VMEM((T, PER_B), jnp.int32),
            pltpu.VMEM((PER_B, EMBED_DIM), jnp.float32),
            pltpu.VMEM((PER_B, EMBED_DIM), jnp.float32),
            pltpu.VMEM((EMBED_DIM, PER_B), jnp.float32),
            pltpu.VMEM((EMBED_DIM, PER_B), jnp.float32),
            pltpu.SemaphoreType.DMA,
            pltpu.SemaphoreType.DMA,
            pltpu.SemaphoreType.DMA,
            pltpu.SemaphoreType.DMA,
        ],
        compiler_params=pltpu.CompilerParams(
            use_tc_tiling_on_sc=False, needs_layout_passes=False
        ),
    )
    return kern(idx_t, table)


@jax.jit
def _embed(idx_t, table):
    g = _gather_sc(idx_t, table)
    return jnp.transpose(g, (2, 0, 1))


def kernel(action_tokens, table):
    idx_t = action_tokens.T.astype(jnp.int32)
    return _embed(idx_t, table)


# speedup vs baseline: 1.3423x; 1.3423x over previous
import jax
import jax.numpy as jnp
from jax import lax
from jax.experimental import pallas as pl
from jax.experimental.pallas import tpu as pltpu
from jax.experimental.pallas import tpu_sc as plsc

VOCAB = 100000
EMBED_DIM = 64
B = 4096
T = 200
N = B * T

NC = 2
NS = 16
NW = NC * NS

PER_B = B // NW
L = 16
FLAT = PER_B * EMBED_DIM


def _fire_gather(table_hbm, idx_v, rows, sem, t):
    pltpu.async_copy(table_hbm.at[idx_v.at[t]], rows, sem)


def _wait_rows(table_hbm, rows, sem):
    pltpu.make_async_copy(table_hbm.at[pl.ds(0, PER_B)], rows, sem).wait()


def _fire_out(out_hbm, tbuf, sem, b0, t):
    pltpu.async_copy(tbuf, out_hbm.at[t, :, pl.ds(b0, PER_B)], sem)


def _wait_out(out_hbm, tbuf, sem):
    pltpu.make_async_copy(tbuf, out_hbm.at[0, :, pl.ds(0, PER_B)], sem).wait()


def _transpose(rows, tbuf, iota64x):
    iota16, zeros16 = iota64x
    for d in range(EMBED_DIM):
        cidx = zeros16 + d
        vs = []
        for jb in range(PER_B // L):
            vs.append(
                plsc.load_gather(rows.at[pl.ds(jb * L, L), :], [iota16, cidx])
            )
        for jb in range(PER_B // L):
            tbuf[d, pl.ds(jb * L, L)] = vs[jb]


def _body(idxt_hbm, table_hbm, out_hbm,
          idx_v, rows0, rows1, tb0, tb1, g0, g1, o0, o1):
    wid = lax.axis_index("s") * NC + lax.axis_index("c")
    b0 = wid * PER_B
    rows = (rows0, rows1)
    tbuf = (tb0, tb1)
    gsem = (g0, g1)
    osem = (o0, o1)
    iota64x = (lax.iota(jnp.int32, L), jnp.zeros((L,), jnp.int32))

    pltpu.sync_copy(idxt_hbm.at[:, pl.ds(b0, PER_B)], idx_v)

    _fire_gather(table_hbm, idx_v, rows[0], gsem[0], 0)
    _fire_gather(table_hbm, idx_v, rows[1], gsem[1], 1)

    def slot(t, b):
        @pl.when(t >= 2)
        def _():
            _wait_out(out_hbm, tbuf[b], osem[b])

        _wait_rows(table_hbm, rows[b], gsem[b])
        _transpose(rows[b], tbuf[b], iota64x)
        _fire_out(out_hbm, tbuf[b], osem[b], b0, t)

        @pl.when(t + 2 < T)
        def _():
            _fire_gather(table_hbm, idx_v, rows[b], gsem[b], t + 2)

    def pair(g, carry):
        slot(2 * g, 0)
        slot(2 * g + 1, 1)
        return carry

    lax.fori_loop(0, T // 2, pair, 0)

    _wait_out(out_hbm, tbuf[0], osem[0])
    _wait_out(out_hbm, tbuf[1], osem[1])


def _gather_sc(idx_t, table):
    mesh = plsc.VectorSubcoreMesh(core_axis_name="c", subcore_axis_name="s")
    kern = pl.kernel(
        _body,
        out_type=jax.ShapeDtypeStruct((T, EMBED_DIM, B), jnp.float32),
        mesh=mesh,
        scratch_types=[
            pltpu.VMEM((T, PER_B), jnp.int32),
            pltpu.VMEM((PER_B, EMBED_DIM), jnp.float32),
            pltpu.VMEM((PER_B, EMBED_DIM), jnp.float32),
            pltpu.VMEM((EMBED_DIM, PER_B), jnp.float32),
            pltpu.VMEM((EMBED_DIM, PER_B), jnp.float32),
            pltpu.SemaphoreType.DMA,
            pltpu.SemaphoreType.DMA,
            pltpu.SemaphoreType.DMA,
            pltpu.SemaphoreType.DMA,
        ],
        compiler_params=pltpu.CompilerParams(
            use_tc_tiling_on_sc=False, needs_layout_passes=False
        ),
    )
    return kern(idx_t, table)


@jax.jit
def _embed(idx_t, table):
    g = _gather_sc(idx_t, table)
    return jnp.transpose(g, (2, 0, 1))


def kernel(action_tokens, table):
    idx_t = action_tokens.T.astype(jnp.int32)
    return _embed(idx_t, table)


# speedup vs baseline: 2.4263x; 1.8076x over previous
import jax
import jax.numpy as jnp
from jax import lax
from jax.experimental import pallas as pl
from jax.experimental.pallas import tpu as pltpu
from jax.experimental.pallas import tpu_sc as plsc

VOCAB = 100000
EMBED_DIM = 64
B = 4096
T = 200
N = B * T

NC = 2
NS = 16
NW = NC * NS

PER_W = N // NW
CHUNK = 512
STEPS = PER_W // CHUNK


def _fire_gather(table_hbm, idx_v, rows, sem, chunk_i):
    pltpu.async_copy(
        table_hbm.at[idx_v.at[pl.ds(chunk_i * CHUNK, CHUNK)]], rows, sem
    )


def _wait_rows(table_hbm, rows, sem):
    pltpu.make_async_copy(table_hbm.at[pl.ds(0, CHUNK)], rows, sem).wait()


def _fire_out(out_hbm, rows, sem, w_base, chunk_i):
    pltpu.async_copy(rows, out_hbm.at[pl.ds(w_base + chunk_i * CHUNK, CHUNK)], sem)


def _wait_out(out_hbm, rows, sem):
    pltpu.make_async_copy(rows, out_hbm.at[pl.ds(0, CHUNK)], sem).wait()


def _body(idx_hbm, table_hbm, out_hbm, idx_v, rows0, rows1, g0, g1, o0, o1):
    wid = lax.axis_index("s") * NC + lax.axis_index("c")
    w_base = wid * PER_W
    rows = (rows0, rows1)
    gsem = (g0, g1)
    osem = (o0, o1)

    pltpu.sync_copy(idx_hbm.at[pl.ds(w_base, PER_W)], idx_v)

    _fire_gather(table_hbm, idx_v, rows[0], gsem[0], 0)
    _wait_rows(table_hbm, rows[0], gsem[0])
    _fire_out(out_hbm, rows[0], osem[0], w_base, 0)
    _fire_gather(table_hbm, idx_v, rows[1], gsem[1], 1)

    def slot(i, b):
        _wait_rows(table_hbm, rows[b], gsem[b])
        _fire_out(out_hbm, rows[b], osem[b], w_base, i)
        _wait_out(out_hbm, rows[1 - b], osem[1 - b])
        _fire_gather(table_hbm, idx_v, rows[1 - b], gsem[1 - b], i + 1)

    def pair(g, carry):
        slot(1 + 2 * g, 1)
        slot(2 + 2 * g, 0)
        return carry

    lax.fori_loop(0, (STEPS - 2) // 2, pair, 0)

    bl = (STEPS - 1) % 2
    _wait_rows(table_hbm, rows[bl], gsem[bl])
    _fire_out(out_hbm, rows[bl], osem[bl], w_base, STEPS - 1)
    _wait_out(out_hbm, rows[1 - bl], osem[1 - bl])
    _wait_out(out_hbm, rows[bl], osem[bl])


@jax.jit
def _embed(idx_flat, table):
    mesh = plsc.VectorSubcoreMesh(core_axis_name="c", subcore_axis_name="s")
    kern = pl.kernel(
        _body,
        out_type=jax.ShapeDtypeStruct((N, EMBED_DIM), jnp.float32),
        mesh=mesh,
        scratch_types=[
            pltpu.VMEM((PER_W,), jnp.int32),
            pltpu.VMEM((CHUNK, EMBED_DIM), jnp.float32),
            pltpu.VMEM((CHUNK, EMBED_DIM), jnp.float32),
            pltpu.SemaphoreType.DMA,
            pltpu.SemaphoreType.DMA,
            pltpu.SemaphoreType.DMA,
            pltpu.SemaphoreType.DMA,
        ],
        compiler_params=pltpu.CompilerParams(use_tc_tiling_on_sc=False),
    )
    return kern(idx_flat, table)


def kernel(action_tokens, table):
    idx_flat = action_tokens.reshape(-1).astype(jnp.int32)
    out = _embed(idx_flat, table)
    return out.reshape(B, T, EMBED_DIM)
